# baseline (speedup 1.0000x reference)
import functools

import jax
import jax.numpy as jnp
from jax import lax
from jax.experimental import pallas as pl
from jax.experimental.pallas import tpu as pltpu
from jax.experimental.pallas import tpu_sc as plsc

N = 10000
D = 128
E = 320000
NC = 2
NS = 16
NW = NC * NS
E_PER_TILE = E // NW
K = 80
CHUNKS = E_PER_TILE // K
R_MAIN = 624
ZR = 48
NBUF = 3
NIDX = 6


def _sc_body(x_hbm, adj_hbm, part_hbm,
             sidx_v, didx_v, rows_v, zero_v, acc_sh, sem_z, *sems):
    sem_i = sems[:NIDX]
    sem_g = sems[NIDX:NIDX + NBUF]
    sem_s = sems[NIDX + NBUF:]
    c = lax.axis_index("c")
    s = lax.axis_index("s")

    wid = c * NS + s

    for r in range(ZR):
        for j in range(D // 16):
            zero_v[r, pl.ds(j * 16, 16)] = jnp.zeros((16,), jnp.float32)

    zcopies = []
    for i in range(R_MAIN // ZR):
        zcopies.append(pltpu.async_copy(
            zero_v, acc_sh.at[pl.ds(s * R_MAIN + i * ZR, ZR)], sem_z))

    @pl.when(s == NS - 1)
    def _zero_tail():
        pltpu.async_copy(zero_v, acc_sh.at[pl.ds(N - ZR, ZR)], sem_z).wait()

    for cp in zcopies:
        cp.wait()
    plsc.subcore_barrier()

    base_e = wid * E_PER_TILE

    def idx_fire(chunk, q):
        off = base_e + chunk * K
        pltpu.async_copy(adj_hbm.at[pl.ds(off, K)], sidx_v.at[q], sem_i[q])
        pltpu.async_copy(adj_hbm.at[pl.ds(E + off, K)], didx_v.at[q],
                         sem_i[q])

    def idx_wait(q):
        pltpu.make_async_copy(adj_hbm.at[pl.ds(0, K)], sidx_v.at[q],
                              sem_i[q]).wait()
        pltpu.make_async_copy(adj_hbm.at[pl.ds(0, K)], didx_v.at[q],
                              sem_i[q]).wait()

    def gather_fire(q, b):
        pltpu.async_copy(x_hbm.at[sidx_v.at[q]], rows_v.at[b], sem_g[b])

    def gather_wait(b):
        pltpu.make_async_copy(x_hbm.at[sidx_v.at[0]], rows_v.at[b],
                              sem_g[b]).wait()

    def scatter_fire(q, b):
        pltpu.async_copy(rows_v.at[b], acc_sh.at[didx_v.at[q]],
                         sem_s[b], add=True)

    def scatter_wait(b):
        pltpu.make_async_copy(rows_v.at[b], acc_sh.at[didx_v.at[0]],
                              sem_s[b]).wait()

    def step(cexpr, u, first=False, fire_idx=True):
        b = u % NBUF
        q = u % NIDX
        if not first:
            scatter_wait(b)
        if fire_idx:
            idx_fire(cexpr + NBUF, (u + NBUF) % NIDX)
        idx_wait(q)
        gather_fire(q, b)
        if not (first and u == 0):
            gather_wait((u - 1) % NBUF)
            scatter_fire((u - 1) % NIDX, (u - 1) % NBUF)

    for q in range(NBUF):
        idx_fire(q, q)
    for u in range(NIDX):
        step(u, u, first=(u < NBUF))

    def ring(j, carry):
        c0 = j * NIDX
        for u in range(NIDX):
            step(c0 + u, u)
        return carry
    lax.fori_loop(1, CHUNKS // NIDX, ring, 0)

    main_end = (CHUNKS // NIDX) * NIDX
    for cc in range(main_end, CHUNKS):
        step(cc, cc % NIDX, fire_idx=(cc + NBUF < CHUNKS))
    last = CHUNKS - 1
    gather_wait(last % NBUF)
    scatter_fire(last % NIDX, last % NBUF)
    for k in range(CHUNKS - NBUF, CHUNKS):
        scatter_wait(k % NBUF)

    plsc.subcore_barrier()
    pltpu.sync_copy(
        acc_sh.at[pl.ds(s * R_MAIN, R_MAIN)],
        part_hbm.at[pl.ds(c * N + s * R_MAIN, R_MAIN)])

    @pl.when(s == NS - 1)
    def _write_tail():
        pltpu.sync_copy(acc_sh.at[pl.ds(N - ZR, ZR)],
                        part_hbm.at[pl.ds(c * N + N - ZR, ZR)])


_sc_aggregate = functools.partial(
    pl.kernel,
    mesh=plsc.VectorSubcoreMesh(core_axis_name="c", subcore_axis_name="s"),
    out_type=jax.ShapeDtypeStruct((NC * N, D), jnp.float32),
    scratch_types=[
        pltpu.VMEM((NIDX, K), jnp.int32),
        pltpu.VMEM((NIDX, K), jnp.int32),
        pltpu.VMEM((NBUF, K, D), jnp.float32),
        pltpu.VMEM((ZR, D), jnp.float32),
        pltpu.VMEM_SHARED((N, D), jnp.float32),
    ] + [pltpu.SemaphoreType.DMA] * (1 + NIDX + 2 * NBUF),
)(_sc_body)


def _mm_body(p0_ref, p1_ref, w_ref, b_ref, o_ref):
    acc = p0_ref[...] + p1_ref[...]
    o_ref[...] = (
        jnp.dot(acc, w_ref[...], preferred_element_type=jnp.float32)
        + b_ref[...]
    )


_BM = 2000


def _mm(partials, W, b2d):
    return pl.pallas_call(
        _mm_body,
        grid=(N // _BM,),
        in_specs=[
            pl.BlockSpec((_BM, D), lambda i: (i, 0)),
            pl.BlockSpec((_BM, D), lambda i: (i + N // _BM, 0)),
            pl.BlockSpec((D, D), lambda i: (0, 0)),
            pl.BlockSpec((1, D), lambda i: (0, 0)),
        ],
        out_specs=pl.BlockSpec((_BM, D), lambda i: (i, 0)),
        out_shape=jax.ShapeDtypeStruct((N, D), jnp.float32),
    )(partials, partials, W, b2d)


def kernel(x, adj_t, W, b):
    partials = _sc_aggregate(x, adj_t.reshape(2 * E))
    return _mm(partials, W, b.reshape(1, D))


# speedup vs baseline: 1.0308x; 1.0308x over previous
import functools

import jax
import jax.numpy as jnp
from jax import lax
from jax.experimental import pallas as pl
from jax.experimental.pallas import tpu as pltpu
from jax.experimental.pallas import tpu_sc as plsc

N = 10000
D = 128
E = 320000
NC = 2
NS = 16
NW = NC * NS
E_PER_TILE = E // NW
K = 80
CHUNKS = E_PER_TILE // K
R_MAIN = 624
ZR = 48
NBUF = 3
NIDX = 6


def _sc_body(x_hbm, adj_hbm, part_hbm,
             sidx_v, didx_v, rows_v, zero_v, acc_sh, sem_z, *sems):
    sem_i = sems[:NIDX]
    sem_g = sems[NIDX:NIDX + NBUF]
    sem_s = sems[NIDX + NBUF:]
    c = lax.axis_index("c")
    s = lax.axis_index("s")

    wid = c * NS + s

    for r in range(ZR):
        for j in range(D // 16):
            zero_v[r, pl.ds(j * 16, 16)] = jnp.zeros((16,), jnp.float32)

    zcopies = []
    for i in range(R_MAIN // ZR):
        zcopies.append(pltpu.async_copy(
            zero_v, acc_sh.at[pl.ds(s * R_MAIN + i * ZR, ZR)], sem_z))

    @pl.when(s == NS - 1)
    def _zero_tail():
        pltpu.async_copy(zero_v, acc_sh.at[pl.ds(N - ZR, ZR)], sem_z).wait()

    for cp in zcopies:
        cp.wait()
    plsc.subcore_barrier()

    base_e = wid * E_PER_TILE

    def idx_fire(chunk, q):
        off = base_e + chunk * K
        pltpu.async_copy(adj_hbm.at[pl.ds(off, K)], sidx_v.at[q], sem_i[q])
        pltpu.async_copy(adj_hbm.at[pl.ds(E + off, K)], didx_v.at[q],
                         sem_i[q])

    def idx_wait(q):
        pltpu.make_async_copy(adj_hbm.at[pl.ds(0, K)], sidx_v.at[q],
                              sem_i[q]).wait()
        pltpu.make_async_copy(adj_hbm.at[pl.ds(0, K)], didx_v.at[q],
                              sem_i[q]).wait()

    def gather_fire(q, b):
        pltpu.async_copy(x_hbm.at[sidx_v.at[q]], rows_v.at[b], sem_g[b])

    def gather_wait(b):
        pltpu.make_async_copy(x_hbm.at[sidx_v.at[0]], rows_v.at[b],
                              sem_g[b]).wait()

    def scatter_fire(q, b):
        pltpu.async_copy(rows_v.at[b], acc_sh.at[didx_v.at[q]],
                         sem_s[b], add=True)

    def scatter_wait(b):
        pltpu.make_async_copy(rows_v.at[b], acc_sh.at[didx_v.at[0]],
                              sem_s[b]).wait()

    def step(cexpr, u, first=False, fire_idx=True):
        b = u % NBUF
        q = u % NIDX
        if not first:
            scatter_wait(b)
        if fire_idx:
            idx_fire(cexpr + NBUF, (u + NBUF) % NIDX)
        idx_wait(q)
        gather_fire(q, b)
        if not (first and u < 2):
            gather_wait((u - 2) % NBUF)
            scatter_fire((u - 2) % NIDX, (u - 2) % NBUF)

    for q in range(NBUF):
        idx_fire(q, q)
    for u in range(NIDX):
        step(u, u, first=(u < NBUF))

    def ring(j, carry):
        c0 = j * NIDX
        for u in range(NIDX):
            step(c0 + u, u)
        return carry
    lax.fori_loop(1, CHUNKS // NIDX, ring, 0)

    main_end = (CHUNKS // NIDX) * NIDX
    for cc in range(main_end, CHUNKS):
        step(cc, cc % NIDX, fire_idx=(cc + NBUF < CHUNKS))
    for cc in (CHUNKS - 2, CHUNKS - 1):
        gather_wait(cc % NBUF)
        scatter_fire(cc % NIDX, cc % NBUF)
    for k in range(CHUNKS - NBUF, CHUNKS):
        scatter_wait(k % NBUF)

    plsc.subcore_barrier()
    pltpu.sync_copy(
        acc_sh.at[pl.ds(s * R_MAIN, R_MAIN)],
        part_hbm.at[pl.ds(c * N + s * R_MAIN, R_MAIN)])

    @pl.when(s == NS - 1)
    def _write_tail():
        pltpu.sync_copy(acc_sh.at[pl.ds(N - ZR, ZR)],
                        part_hbm.at[pl.ds(c * N + N - ZR, ZR)])


_sc_aggregate = functools.partial(
    pl.kernel,
    mesh=plsc.VectorSubcoreMesh(core_axis_name="c", subcore_axis_name="s"),
    out_type=jax.ShapeDtypeStruct((NC * N, D), jnp.float32),
    scratch_types=[
        pltpu.VMEM((NIDX, K), jnp.int32),
        pltpu.VMEM((NIDX, K), jnp.int32),
        pltpu.VMEM((NBUF, K, D), jnp.float32),
        pltpu.VMEM((ZR, D), jnp.float32),
        pltpu.VMEM_SHARED((N, D), jnp.float32),
    ] + [pltpu.SemaphoreType.DMA] * (1 + NIDX + 2 * NBUF),
)(_sc_body)


def _mm_body(p0_ref, p1_ref, w_ref, b_ref, o_ref):
    acc = p0_ref[...] + p1_ref[...]
    o_ref[...] = (
        jnp.dot(acc, w_ref[...], preferred_element_type=jnp.float32)
        + b_ref[...]
    )


_BM = 2000


def _mm(partials, W, b2d):
    return pl.pallas_call(
        _mm_body,
        grid=(N // _BM,),
        in_specs=[
            pl.BlockSpec((_BM, D), lambda i: (i, 0)),
            pl.BlockSpec((_BM, D), lambda i: (i + N // _BM, 0)),
            pl.BlockSpec((D, D), lambda i: (0, 0)),
            pl.BlockSpec((1, D), lambda i: (0, 0)),
        ],
        out_specs=pl.BlockSpec((_BM, D), lambda i: (i, 0)),
        out_shape=jax.ShapeDtypeStruct((N, D), jnp.float32),
    )(partials, partials, W, b2d)


def kernel(x, adj_t, W, b):
    partials = _sc_aggregate(x, adj_t.reshape(2 * E))
    return _mm(partials, W, b.reshape(1, D))
